# baseline (speedup 1.0000x reference)
import jax
import jax.numpy as jnp
from jax import lax
from jax.experimental import pallas as pl
from jax.experimental.pallas import tpu as pltpu
from jax.experimental.pallas import tpu_sc as plsc

N_NODES = 10000
D = 128
D_OUT = 40
E_RAW = 320000

NC, NS = 2, 16
NW = NC * NS
K = 128
E_TOT = E_RAW + N_NODES
CHSUM = 168
CH0 = 64
CH1 = CHSUM - CH0
CHMAX = max(CH0, CH1)
E_PAD = NS * CHSUM * K
N_PAD = 10240
RPT = N_PAD // NS
BLK = 512
_MESH = plsc.VectorSubcoreMesh(core_axis_name="c", subcore_axis_name="s")


def _load_my_chunks(hbm, vmem, c, s):
    @pl.when(c == 0)
    def _():
        pltpu.sync_copy(hbm.at[c, s, pl.ds(0, CH0)], vmem.at[pl.ds(0, CH0)])

    @pl.when(c != 0)
    def _():
        pltpu.sync_copy(hbm.at[c, s, pl.ds(0, CH1)], vmem.at[pl.ds(0, CH1)])

    return jnp.where(c == 0, CH0, CH1)


def _deg_body(dsts_hbm, ones_hbm, zeros_hbm, out_hbm, idx_d, ones_v, acc):
    c = lax.axis_index("c")
    s = lax.axis_index("s")
    cnt = _load_my_chunks(dsts_hbm, idx_d, c, s)
    pltpu.sync_copy(ones_hbm, ones_v)
    pltpu.sync_copy(zeros_hbm, acc.at[pl.ds(s * RPT, RPT)])
    plsc.subcore_barrier()

    def chunk(j, carry):
        pltpu.sync_copy(ones_v, acc.at[idx_d.at[j]], add=True)
        return carry

    lax.fori_loop(0, cnt, chunk, 0)
    plsc.subcore_barrier()
    pltpu.sync_copy(acc.at[pl.ds(s * RPT, RPT)],
                    out_hbm.at[c, pl.ds(s * RPT, RPT)])


_deg_call = pl.kernel(
    _deg_body,
    out_type=jax.ShapeDtypeStruct((NC, N_PAD, D), jnp.float32),
    mesh=_MESH,
    scratch_types=[
        pltpu.VMEM((CHMAX, K), jnp.int32),
        pltpu.VMEM((K, D), jnp.float32),
        pltpu.VMEM_SHARED((N_PAD, D), jnp.float32),
    ],
)


def _seg_body(g_hbm, srcs_hbm, dsts_hbm, zeros_hbm, out_hbm,
              idx_s, idx_d, rows, acc, gsem, isem, ssem):
    c = lax.axis_index("c")
    s = lax.axis_index("s")
    cnt = _load_my_chunks(dsts_hbm, idx_d, c, s)
    pltpu.sync_copy(srcs_hbm.at[c, s, 0], idx_s.at[0])
    pltpu.async_copy(g_hbm.at[idx_s.at[0]], rows.at[0], gsem)
    pltpu.async_copy(srcs_hbm.at[c, s, 1], idx_s.at[1], isem)
    pltpu.sync_copy(zeros_hbm, acc.at[pl.ds(s * RPT, RPT)])
    plsc.subcore_barrier()

    def chunk(j, carry):
        b = lax.rem(j, 2)
        pltpu.make_async_copy(g_hbm.at[idx_s.at[0]], rows.at[b], gsem).wait()

        @pl.when(j >= 1)
        def _():
            pltpu.make_async_copy(rows.at[0], acc.at[idx_d.at[0]], ssem).wait()

        @pl.when(j + 1 < cnt)
        def _():
            pltpu.make_async_copy(srcs_hbm.at[c, s, 0], idx_s.at[0], isem).wait()
            pltpu.async_copy(g_hbm.at[idx_s.at[lax.rem(j + 1, 3)]],
                             rows.at[1 - b], gsem)

        @pl.when(j + 2 < cnt)
        def _():
            pltpu.async_copy(srcs_hbm.at[c, s, j + 2],
                             idx_s.at[lax.rem(j + 2, 3)], isem)

        pltpu.async_copy(rows.at[b], acc.at[idx_d.at[j]], ssem, add=True)
        return carry

    lax.fori_loop(0, cnt, chunk, 0)
    pltpu.make_async_copy(rows.at[0], acc.at[idx_d.at[0]], ssem).wait()
    plsc.subcore_barrier()
    pltpu.sync_copy(acc.at[pl.ds(s * RPT, RPT)],
                    out_hbm.at[c, pl.ds(s * RPT, RPT)])


_seg_call = pl.kernel(
    _seg_body,
    out_type=jax.ShapeDtypeStruct((NC, N_PAD, D), jnp.float32),
    mesh=_MESH,
    scratch_types=[
        pltpu.VMEM((3, K), jnp.int32),
        pltpu.VMEM((CHMAX, K), jnp.int32),
        pltpu.VMEM((2, K, D), jnp.float32),
        pltpu.VMEM_SHARED((N_PAD, D), jnp.float32),
        pltpu.SemaphoreType.DMA,
        pltpu.SemaphoreType.DMA,
        pltpu.SemaphoreType.DMA,
    ],
)


def _tc_mm_body(x_ref, w_ref, o_ref):
    o_ref[...] = jnp.dot(x_ref[...], w_ref[...],
                         preferred_element_type=jnp.float32)


_tc_mm = pl.pallas_call(
    _tc_mm_body,
    grid=(N_PAD // BLK,),
    in_specs=[
        pl.BlockSpec((BLK, D), lambda i: (i, 0)),
        pl.BlockSpec((D, D), lambda i: (0, 0)),
    ],
    out_specs=pl.BlockSpec((BLK, D), lambda i: (i, 0)),
    out_shape=jax.ShapeDtypeStruct((N_PAD, D), jnp.float32),
)


def _tc_dis_body(deg_ref, hw_ref, dis_ref, g_ref):
    deg = deg_ref[0] + deg_ref[1]
    dis = jnp.where(deg > 0, lax.rsqrt(deg), 0.0)
    dis_ref[...] = dis
    g_ref[...] = hw_ref[...] * dis


_tc_dis = pl.pallas_call(
    _tc_dis_body,
    grid=(N_PAD // BLK,),
    in_specs=[
        pl.BlockSpec((NC, BLK, D), lambda i: (0, i, 0)),
        pl.BlockSpec((BLK, D), lambda i: (i, 0)),
    ],
    out_specs=[
        pl.BlockSpec((BLK, D), lambda i: (i, 0)),
        pl.BlockSpec((BLK, D), lambda i: (i, 0)),
    ],
    out_shape=[
        jax.ShapeDtypeStruct((N_PAD, D), jnp.float32),
        jax.ShapeDtypeStruct((N_PAD, D), jnp.float32),
    ],
)


def _tc_b_body(acc_ref, dis_ref, b_ref, w_ref, g_ref):
    dis = dis_ref[...]
    h = jnp.maximum(dis * (acc_ref[0] + acc_ref[1]) + b_ref[...], 0.0)
    g_ref[...] = jnp.dot(h, w_ref[...],
                         preferred_element_type=jnp.float32) * dis


_tc_b = pl.pallas_call(
    _tc_b_body,
    grid=(N_PAD // BLK,),
    in_specs=[
        pl.BlockSpec((NC, BLK, D), lambda i: (0, i, 0)),
        pl.BlockSpec((BLK, D), lambda i: (i, 0)),
        pl.BlockSpec((D,), lambda i: (0,)),
        pl.BlockSpec((D, D), lambda i: (0, 0)),
    ],
    out_specs=pl.BlockSpec((BLK, D), lambda i: (i, 0)),
    out_shape=jax.ShapeDtypeStruct((N_PAD, D), jnp.float32),
)


def _tc_c_body(acc_ref, dis_ref, b_ref, w_ref, bfc_ref, out_ref):
    dis = dis_ref[...]
    h = jnp.maximum(dis * (acc_ref[0] + acc_ref[1]) + b_ref[...], 0.0)
    out_ref[...] = jnp.dot(h, w_ref[...],
                           preferred_element_type=jnp.float32) + bfc_ref[...][None, :]


_tc_c = pl.pallas_call(
    _tc_c_body,
    grid=(N_PAD // BLK,),
    in_specs=[
        pl.BlockSpec((NC, BLK, D), lambda i: (0, i, 0)),
        pl.BlockSpec((BLK, D), lambda i: (i, 0)),
        pl.BlockSpec((D,), lambda i: (0,)),
        pl.BlockSpec((D, D), lambda i: (0, 0)),
        pl.BlockSpec((D,), lambda i: (0,)),
    ],
    out_specs=pl.BlockSpec((BLK, D), lambda i: (i, 0)),
    out_shape=jax.ShapeDtypeStruct((N_PAD, D), jnp.float32),
)


def kernel(x, edge_index, W0, b0, W1, b1, W2, b2, Wfc, bfc):
    n = x.shape[0]
    idt = edge_index.dtype
    loop = jnp.arange(n, dtype=idt)
    pad = jnp.full((E_PAD - E_TOT,), n, dtype=idt)

    def _chunked(flat):
        p0 = flat[:NS * CH0 * K].reshape(NS, CH0, K)
        p1 = flat[NS * CH0 * K:].reshape(NS, CH1, K)
        arr = jnp.full((NC, NS, CHMAX, K), n, dtype=idt)
        return arr.at[0, :, :CH0].set(p0).at[1, :, :CH1].set(p1)

    srcs = _chunked(jnp.concatenate([edge_index[0], loop, pad]))
    dsts = _chunked(jnp.concatenate([edge_index[1], loop, pad]))

    x_pad = jnp.zeros((N_PAD, D), jnp.float32).at[:n].set(x)
    zeros = jnp.zeros((RPT, D), jnp.float32)
    ones = jnp.ones((K, D), jnp.float32)
    Wfc_p = jnp.zeros((D, D), jnp.float32).at[:, :D_OUT].set(Wfc)
    bfc_p = jnp.zeros((D,), jnp.float32).at[:D_OUT].set(bfc)

    deg2 = _deg_call(dsts, ones, zeros)
    hw0 = _tc_mm(x_pad, W0)
    dis, g = _tc_dis(deg2, hw0)
    acc = _seg_call(g, srcs, dsts, zeros)
    g = _tc_b(acc, dis, b0, W1)
    acc = _seg_call(g, srcs, dsts, zeros)
    g = _tc_b(acc, dis, b1, W2)
    acc = _seg_call(g, srcs, dsts, zeros)
    out = _tc_c(acc, dis, b2, Wfc_p, bfc_p)
    return out[:n, :D_OUT]


# speedup vs baseline: 4.1604x; 4.1604x over previous
import jax
import jax.numpy as jnp
from jax import lax
from jax.experimental import pallas as pl
from jax.experimental.pallas import tpu as pltpu
from jax.experimental.pallas import tpu_sc as plsc

N_NODES = 10000
D = 128
D_OUT = 40
E_RAW = 320000

NC, NS = 2, 16
NW = NC * NS
K = 128
E_TOT = E_RAW + N_NODES
CH = -(-E_TOT // (NW * K))
E_PAD = NW * CH * K
N_PAD = 10240
RPT = N_PAD // NS
BLK = 512
_MESH = plsc.VectorSubcoreMesh(core_axis_name="c", subcore_axis_name="s")


def _deg_body(dsts_hbm, ones_hbm, zeros_hbm, out_hbm, idx_d, ones_v, acc):
    c = lax.axis_index("c")
    s = lax.axis_index("s")
    wid = s * NC + c
    pltpu.sync_copy(dsts_hbm.at[wid], idx_d)
    pltpu.sync_copy(ones_hbm, ones_v)
    pltpu.sync_copy(zeros_hbm, acc.at[pl.ds(s * RPT, RPT)])
    plsc.subcore_barrier()

    def chunk(j, carry):
        pltpu.sync_copy(ones_v, acc.at[idx_d.at[j]], add=True)
        return carry

    lax.fori_loop(0, CH, chunk, 0)
    plsc.subcore_barrier()
    pltpu.sync_copy(acc.at[pl.ds(s * RPT, RPT)],
                    out_hbm.at[c, pl.ds(s * RPT, RPT)])


_deg_call = pl.kernel(
    _deg_body,
    out_type=jax.ShapeDtypeStruct((NC, N_PAD, D), jnp.float32),
    mesh=_MESH,
    scratch_types=[
        pltpu.VMEM((CH, K), jnp.int32),
        pltpu.VMEM((K, D), jnp.float32),
        pltpu.VMEM_SHARED((N_PAD, D), jnp.float32),
    ],
)


def _seg_body(g_hbm, srcs_hbm, dsts_hbm, zeros_hbm, out_hbm,
              idx_s, idx_d, rows, acc, gsem, isem, ssem):
    c = lax.axis_index("c")
    s = lax.axis_index("s")
    wid = s * NC + c
    pltpu.sync_copy(dsts_hbm.at[wid], idx_d)
    pltpu.sync_copy(srcs_hbm.at[wid, 0], idx_s.at[0])
    pltpu.async_copy(g_hbm.at[idx_s.at[0]], rows.at[0], gsem)
    pltpu.async_copy(srcs_hbm.at[wid, 1], idx_s.at[1], isem)
    pltpu.sync_copy(zeros_hbm, acc.at[pl.ds(s * RPT, RPT)])
    plsc.subcore_barrier()

    def chunk(j, carry):
        b = lax.rem(j, 2)
        pltpu.make_async_copy(g_hbm.at[idx_s.at[0]], rows.at[b], gsem).wait()

        @pl.when(j >= 1)
        def _():
            pltpu.make_async_copy(rows.at[0], acc.at[idx_d.at[0]], ssem).wait()

        @pl.when(j + 1 < CH)
        def _():
            pltpu.make_async_copy(srcs_hbm.at[wid, 0], idx_s.at[0], isem).wait()
            pltpu.async_copy(g_hbm.at[idx_s.at[lax.rem(j + 1, 3)]],
                             rows.at[1 - b], gsem)

        @pl.when(j + 2 < CH)
        def _():
            pltpu.async_copy(srcs_hbm.at[wid, j + 2],
                             idx_s.at[lax.rem(j + 2, 3)], isem)

        pltpu.async_copy(rows.at[b], acc.at[idx_d.at[j]], ssem, add=True)
        return carry

    lax.fori_loop(0, CH, chunk, 0)
    pltpu.make_async_copy(rows.at[0], acc.at[idx_d.at[0]], ssem).wait()
    plsc.subcore_barrier()
    pltpu.sync_copy(acc.at[pl.ds(s * RPT, RPT)],
                    out_hbm.at[c, pl.ds(s * RPT, RPT)])


_seg_call = pl.kernel(
    _seg_body,
    out_type=jax.ShapeDtypeStruct((NC, N_PAD, D), jnp.float32),
    mesh=_MESH,
    scratch_types=[
        pltpu.VMEM((3, K), jnp.int32),
        pltpu.VMEM((CH, K), jnp.int32),
        pltpu.VMEM((2, K, D), jnp.float32),
        pltpu.VMEM_SHARED((N_PAD, D), jnp.float32),
        pltpu.SemaphoreType.DMA,
        pltpu.SemaphoreType.DMA,
        pltpu.SemaphoreType.DMA,
    ],
)


def _tc_mm_body(x_ref, w_ref, o_ref):
    o_ref[...] = jnp.dot(x_ref[...], w_ref[...],
                         preferred_element_type=jnp.float32)


_tc_mm = pl.pallas_call(
    _tc_mm_body,
    grid=(N_PAD // BLK,),
    in_specs=[
        pl.BlockSpec((BLK, D), lambda i: (i, 0)),
        pl.BlockSpec((D, D), lambda i: (0, 0)),
    ],
    out_specs=pl.BlockSpec((BLK, D), lambda i: (i, 0)),
    out_shape=jax.ShapeDtypeStruct((N_PAD, D), jnp.float32),
)


def _tc_dis_body(deg_ref, hw_ref, dis_ref, g_ref):
    deg = deg_ref[0] + deg_ref[1]
    dis = jnp.where(deg > 0, lax.rsqrt(deg), 0.0)
    dis_ref[...] = dis
    g_ref[...] = hw_ref[...] * dis


_tc_dis = pl.pallas_call(
    _tc_dis_body,
    grid=(N_PAD // BLK,),
    in_specs=[
        pl.BlockSpec((NC, BLK, D), lambda i: (0, i, 0)),
        pl.BlockSpec((BLK, D), lambda i: (i, 0)),
    ],
    out_specs=[
        pl.BlockSpec((BLK, D), lambda i: (i, 0)),
        pl.BlockSpec((BLK, D), lambda i: (i, 0)),
    ],
    out_shape=[
        jax.ShapeDtypeStruct((N_PAD, D), jnp.float32),
        jax.ShapeDtypeStruct((N_PAD, D), jnp.float32),
    ],
)


def _tc_b_body(acc_ref, dis_ref, b_ref, w_ref, g_ref):
    dis = dis_ref[...]
    h = jnp.maximum(dis * (acc_ref[0] + acc_ref[1]) + b_ref[...], 0.0)
    g_ref[...] = jnp.dot(h, w_ref[...],
                         preferred_element_type=jnp.float32) * dis


_tc_b = pl.pallas_call(
    _tc_b_body,
    grid=(N_PAD // BLK,),
    in_specs=[
        pl.BlockSpec((NC, BLK, D), lambda i: (0, i, 0)),
        pl.BlockSpec((BLK, D), lambda i: (i, 0)),
        pl.BlockSpec((D,), lambda i: (0,)),
        pl.BlockSpec((D, D), lambda i: (0, 0)),
    ],
    out_specs=pl.BlockSpec((BLK, D), lambda i: (i, 0)),
    out_shape=jax.ShapeDtypeStruct((N_PAD, D), jnp.float32),
)


def _tc_c_body(acc_ref, dis_ref, b_ref, w_ref, bfc_ref, out_ref):
    dis = dis_ref[...]
    h = jnp.maximum(dis * (acc_ref[0] + acc_ref[1]) + b_ref[...], 0.0)
    out_ref[...] = jnp.dot(h, w_ref[...],
                           preferred_element_type=jnp.float32) + bfc_ref[...][None, :]


_tc_c = pl.pallas_call(
    _tc_c_body,
    grid=(N_PAD // BLK,),
    in_specs=[
        pl.BlockSpec((NC, BLK, D), lambda i: (0, i, 0)),
        pl.BlockSpec((BLK, D), lambda i: (i, 0)),
        pl.BlockSpec((D,), lambda i: (0,)),
        pl.BlockSpec((D, D), lambda i: (0, 0)),
        pl.BlockSpec((D,), lambda i: (0,)),
    ],
    out_specs=pl.BlockSpec((BLK, D), lambda i: (i, 0)),
    out_shape=jax.ShapeDtypeStruct((N_PAD, D), jnp.float32),
)


def kernel(x, edge_index, W0, b0, W1, b1, W2, b2, Wfc, bfc):
    n = x.shape[0]
    idt = edge_index.dtype
    loop = jnp.arange(n, dtype=idt)
    pad = n + jnp.arange(E_PAD - E_TOT, dtype=idt) % (N_PAD - n)
    srcs = jnp.concatenate([edge_index[0], loop, pad]).reshape(NW, CH, K)
    dsts = jnp.concatenate([edge_index[1], loop, pad]).reshape(NW, CH, K)

    x_pad = jnp.zeros((N_PAD, D), jnp.float32).at[:n].set(x)
    zeros = jnp.zeros((RPT, D), jnp.float32)
    ones = jnp.ones((K, D), jnp.float32)
    Wfc_p = jnp.zeros((D, D), jnp.float32).at[:, :D_OUT].set(Wfc)
    bfc_p = jnp.zeros((D,), jnp.float32).at[:D_OUT].set(bfc)

    deg2 = _deg_call(dsts, ones, zeros)
    hw0 = _tc_mm(x_pad, W0)
    dis, g = _tc_dis(deg2, hw0)
    acc = _seg_call(g, srcs, dsts, zeros)
    g = _tc_b(acc, dis, b0, W1)
    acc = _seg_call(g, srcs, dsts, zeros)
    g = _tc_b(acc, dis, b1, W2)
    acc = _seg_call(g, srcs, dsts, zeros)
    out = _tc_c(acc, dis, b2, Wfc_p, bfc_p)
    return out[:n, :D_OUT]


# speedup vs baseline: 4.5102x; 1.0841x over previous
import jax
import jax.numpy as jnp
from jax import lax
from jax.experimental import pallas as pl
from jax.experimental.pallas import tpu as pltpu
from jax.experimental.pallas import tpu_sc as plsc

N_NODES = 10000
D = 128
D_OUT = 40
E_RAW = 320000

NC, NS = 2, 16
NW = NC * NS
K = 128
E_TOT = E_RAW + N_NODES
CH = -(-E_TOT // (NW * K))
E_PAD = NW * CH * K
N_PAD = 10240
RPT = N_PAD // NS
DEGW = 16
BLK = 512
_MESH = plsc.VectorSubcoreMesh(core_axis_name="c", subcore_axis_name="s")


def _deg_body(dsts_hbm, ones_hbm, zeros_hbm, out_hbm, idx_d, ones_v, acc):
    c = lax.axis_index("c")
    s = lax.axis_index("s")
    wid = s * NC + c
    pltpu.sync_copy(dsts_hbm.at[wid], idx_d)
    pltpu.sync_copy(ones_hbm, ones_v)
    pltpu.sync_copy(zeros_hbm, acc.at[pl.ds(s * RPT, RPT)])
    plsc.subcore_barrier()

    def chunk(j, carry):
        pltpu.sync_copy(ones_v, acc.at[idx_d.at[j]], add=True)
        return carry

    lax.fori_loop(0, CH, chunk, 0)
    plsc.subcore_barrier()
    pltpu.sync_copy(acc.at[pl.ds(s * RPT, RPT)],
                    out_hbm.at[c, pl.ds(s * RPT, RPT)])


_deg_call = pl.kernel(
    _deg_body,
    out_type=jax.ShapeDtypeStruct((NC, N_PAD, DEGW), jnp.float32),
    mesh=_MESH,
    scratch_types=[
        pltpu.VMEM((CH, K), jnp.int32),
        pltpu.VMEM((K, DEGW), jnp.float32),
        pltpu.VMEM_SHARED((N_PAD, DEGW), jnp.float32),
    ],
    compiler_params=pltpu.CompilerParams(use_tc_tiling_on_sc=False),
)


def _seg_body(g_hbm, srcs_hbm, dsts_hbm, zeros_hbm, out_hbm,
              idx_s, idx_d, rows, acc, gsem, isem, ssem):
    c = lax.axis_index("c")
    s = lax.axis_index("s")
    wid = s * NC + c
    pltpu.sync_copy(dsts_hbm.at[wid], idx_d)
    pltpu.sync_copy(srcs_hbm.at[wid, 0], idx_s.at[0])
    pltpu.async_copy(g_hbm.at[idx_s.at[0]], rows.at[0], gsem)
    pltpu.async_copy(srcs_hbm.at[wid, 1], idx_s.at[1], isem)
    pltpu.sync_copy(zeros_hbm, acc.at[pl.ds(s * RPT, RPT)])
    plsc.subcore_barrier()

    def chunk(j, carry):
        b = lax.rem(j, 2)
        pltpu.make_async_copy(g_hbm.at[idx_s.at[0]], rows.at[b], gsem).wait()

        @pl.when(j >= 1)
        def _():
            pltpu.make_async_copy(rows.at[0], acc.at[idx_d.at[0]], ssem).wait()

        @pl.when(j + 1 < CH)
        def _():
            pltpu.make_async_copy(srcs_hbm.at[wid, 0], idx_s.at[0], isem).wait()
            pltpu.async_copy(g_hbm.at[idx_s.at[lax.rem(j + 1, 3)]],
                             rows.at[1 - b], gsem)

        @pl.when(j + 2 < CH)
        def _():
            pltpu.async_copy(srcs_hbm.at[wid, j + 2],
                             idx_s.at[lax.rem(j + 2, 3)], isem)

        pltpu.async_copy(rows.at[b], acc.at[idx_d.at[j]], ssem, add=True)
        return carry

    lax.fori_loop(0, CH, chunk, 0)
    pltpu.make_async_copy(rows.at[0], acc.at[idx_d.at[0]], ssem).wait()
    plsc.subcore_barrier()
    pltpu.sync_copy(acc.at[pl.ds(s * RPT, RPT)],
                    out_hbm.at[c, pl.ds(s * RPT, RPT)])


_seg_call = pl.kernel(
    _seg_body,
    out_type=jax.ShapeDtypeStruct((NC, N_PAD, D), jnp.float32),
    mesh=_MESH,
    scratch_types=[
        pltpu.VMEM((3, K), jnp.int32),
        pltpu.VMEM((CH, K), jnp.int32),
        pltpu.VMEM((2, K, D), jnp.float32),
        pltpu.VMEM_SHARED((N_PAD, D), jnp.float32),
        pltpu.SemaphoreType.DMA,
        pltpu.SemaphoreType.DMA,
        pltpu.SemaphoreType.DMA,
    ],
)


def _tc_mm_body(x_ref, w_ref, o_ref):
    o_ref[...] = jnp.dot(x_ref[...], w_ref[...],
                         preferred_element_type=jnp.float32)


_tc_mm = pl.pallas_call(
    _tc_mm_body,
    grid=(N_PAD // BLK,),
    in_specs=[
        pl.BlockSpec((BLK, D), lambda i: (i, 0)),
        pl.BlockSpec((D, D), lambda i: (0, 0)),
    ],
    out_specs=pl.BlockSpec((BLK, D), lambda i: (i, 0)),
    out_shape=jax.ShapeDtypeStruct((N_PAD, D), jnp.float32),
)


def _tc_dis_body(deg_ref, hw_ref, dis_ref, g_ref):
    deg = deg_ref[0, :, 0] + deg_ref[1, :, 0]
    dis = jnp.where(deg > 0, lax.rsqrt(deg), 0.0)[:, None]
    dis_ref[...] = jnp.broadcast_to(dis, (BLK, D))
    g_ref[...] = hw_ref[...] * dis


_tc_dis = pl.pallas_call(
    _tc_dis_body,
    grid=(N_PAD // BLK,),
    in_specs=[
        pl.BlockSpec((NC, BLK, DEGW), lambda i: (0, i, 0)),
        pl.BlockSpec((BLK, D), lambda i: (i, 0)),
    ],
    out_specs=[
        pl.BlockSpec((BLK, D), lambda i: (i, 0)),
        pl.BlockSpec((BLK, D), lambda i: (i, 0)),
    ],
    out_shape=[
        jax.ShapeDtypeStruct((N_PAD, D), jnp.float32),
        jax.ShapeDtypeStruct((N_PAD, D), jnp.float32),
    ],
)


def _tc_b_body(acc_ref, dis_ref, b_ref, w_ref, g_ref):
    dis = dis_ref[...]
    h = jnp.maximum(dis * (acc_ref[0] + acc_ref[1]) + b_ref[...], 0.0)
    g_ref[...] = jnp.dot(h, w_ref[...],
                         preferred_element_type=jnp.float32) * dis


_tc_b = pl.pallas_call(
    _tc_b_body,
    grid=(N_PAD // BLK,),
    in_specs=[
        pl.BlockSpec((NC, BLK, D), lambda i: (0, i, 0)),
        pl.BlockSpec((BLK, D), lambda i: (i, 0)),
        pl.BlockSpec((D,), lambda i: (0,)),
        pl.BlockSpec((D, D), lambda i: (0, 0)),
    ],
    out_specs=pl.BlockSpec((BLK, D), lambda i: (i, 0)),
    out_shape=jax.ShapeDtypeStruct((N_PAD, D), jnp.float32),
)


def _tc_c_body(acc_ref, dis_ref, b_ref, w_ref, bfc_ref, out_ref):
    dis = dis_ref[...]
    h = jnp.maximum(dis * (acc_ref[0] + acc_ref[1]) + b_ref[...], 0.0)
    out_ref[...] = jnp.dot(h, w_ref[...],
                           preferred_element_type=jnp.float32) + bfc_ref[...][None, :]


_tc_c = pl.pallas_call(
    _tc_c_body,
    grid=(N_PAD // BLK,),
    in_specs=[
        pl.BlockSpec((NC, BLK, D), lambda i: (0, i, 0)),
        pl.BlockSpec((BLK, D), lambda i: (i, 0)),
        pl.BlockSpec((D,), lambda i: (0,)),
        pl.BlockSpec((D, D), lambda i: (0, 0)),
        pl.BlockSpec((D,), lambda i: (0,)),
    ],
    out_specs=pl.BlockSpec((BLK, D), lambda i: (i, 0)),
    out_shape=jax.ShapeDtypeStruct((N_PAD, D), jnp.float32),
)


def kernel(x, edge_index, W0, b0, W1, b1, W2, b2, Wfc, bfc):
    n = x.shape[0]
    idt = edge_index.dtype
    loop = jnp.arange(n, dtype=idt)
    pad = n + jnp.arange(E_PAD - E_TOT, dtype=idt) % (N_PAD - n)
    srcs = jnp.concatenate([edge_index[0], loop, pad]).reshape(NW, CH, K)
    dsts = jnp.concatenate([edge_index[1], loop, pad]).reshape(NW, CH, K)

    x_pad = jnp.zeros((N_PAD, D), jnp.float32).at[:n].set(x)
    zeros = jnp.zeros((RPT, D), jnp.float32)
    zeros_d = jnp.zeros((RPT, DEGW), jnp.float32)
    ones = jnp.ones((K, DEGW), jnp.float32)
    Wfc_p = jnp.zeros((D, D), jnp.float32).at[:, :D_OUT].set(Wfc)
    bfc_p = jnp.zeros((D,), jnp.float32).at[:D_OUT].set(bfc)

    deg2 = _deg_call(dsts, ones, zeros_d)
    hw0 = _tc_mm(x_pad, W0)
    dis, g = _tc_dis(deg2, hw0)
    acc = _seg_call(g, srcs, dsts, zeros)
    g = _tc_b(acc, dis, b0, W1)
    acc = _seg_call(g, srcs, dsts, zeros)
    g = _tc_b(acc, dis, b1, W2)
    acc = _seg_call(g, srcs, dsts, zeros)
    out = _tc_c(acc, dis, b2, Wfc_p, bfc_p)
    return out[:n, :D_OUT]


# speedup vs baseline: 4.6074x; 1.0215x over previous
import jax
import jax.numpy as jnp
from jax import lax
from jax.experimental import pallas as pl
from jax.experimental.pallas import tpu as pltpu
from jax.experimental.pallas import tpu_sc as plsc

N_NODES = 10000
D = 128
D_OUT = 40
E_RAW = 320000

NC, NS = 2, 16
NW = NC * NS
K = 128
E_TOT = E_RAW + N_NODES
CH = -(-E_TOT // (NW * K))
E_PAD = NW * CH * K
N_PAD = 10240
RPT = N_PAD // NS
DEGW = 16
BLK = 512
_MESH = plsc.VectorSubcoreMesh(core_axis_name="c", subcore_axis_name="s")


def _deg_body(dsts_hbm, ones_hbm, zeros_hbm, out_hbm, idx_d, ones_v, acc, dsem):
    c = lax.axis_index("c")
    s = lax.axis_index("s")
    wid = s * NC + c
    pltpu.sync_copy(dsts_hbm.at[wid], idx_d)
    pltpu.sync_copy(ones_hbm, ones_v)
    pltpu.sync_copy(zeros_hbm, acc.at[pl.ds(s * RPT, RPT)])
    plsc.subcore_barrier()

    def group(gi, carry):
        for k in range(8):
            pltpu.async_copy(ones_v, acc.at[idx_d.at[gi * 8 + k]], dsem,
                             add=True)
        for k in range(8):
            pltpu.make_async_copy(ones_v, acc.at[idx_d.at[0]], dsem).wait()
        return carry

    lax.fori_loop(0, CH // 8, group, 0)
    for j in range(CH - CH % 8, CH):
        pltpu.sync_copy(ones_v, acc.at[idx_d.at[j]], add=True)
    plsc.subcore_barrier()
    pltpu.sync_copy(acc.at[pl.ds(s * RPT, RPT)],
                    out_hbm.at[c, pl.ds(s * RPT, RPT)])


_deg_call = pl.kernel(
    _deg_body,
    out_type=jax.ShapeDtypeStruct((NC, N_PAD, DEGW), jnp.float32),
    mesh=_MESH,
    scratch_types=[
        pltpu.VMEM((CH, K), jnp.int32),
        pltpu.VMEM((K, DEGW), jnp.float32),
        pltpu.VMEM_SHARED((N_PAD, DEGW), jnp.float32),
        pltpu.SemaphoreType.DMA,
    ],
    compiler_params=pltpu.CompilerParams(use_tc_tiling_on_sc=False),
)


def _seg_body(g_hbm, srcs_hbm, dsts_hbm, zeros_hbm, out_hbm,
              idx_s, idx_d, rows, acc, gsem, isem, ssem):
    c = lax.axis_index("c")
    s = lax.axis_index("s")
    wid = s * NC + c
    pltpu.sync_copy(dsts_hbm.at[wid], idx_d)
    pltpu.sync_copy(srcs_hbm.at[wid, 0], idx_s.at[0])
    pltpu.async_copy(g_hbm.at[idx_s.at[0]], rows.at[0], gsem)
    pltpu.async_copy(srcs_hbm.at[wid, 1], idx_s.at[1], isem)
    pltpu.sync_copy(zeros_hbm, acc.at[pl.ds(s * RPT, RPT)])
    plsc.subcore_barrier()

    def chunk(j, carry):
        b = lax.rem(j, 2)
        pltpu.make_async_copy(g_hbm.at[idx_s.at[0]], rows.at[b], gsem).wait()
        pltpu.async_copy(rows.at[b], acc.at[idx_d.at[j]], ssem.at[b], add=True)

        @pl.when(j >= 1)
        def _():
            pltpu.make_async_copy(rows.at[0], acc.at[idx_d.at[0]],
                                  ssem.at[1 - b]).wait()

        @pl.when(j + 1 < CH)
        def _():
            pltpu.make_async_copy(srcs_hbm.at[wid, 0], idx_s.at[0], isem).wait()
            pltpu.async_copy(g_hbm.at[idx_s.at[lax.rem(j + 1, 3)]],
                             rows.at[1 - b], gsem)

        @pl.when(j + 2 < CH)
        def _():
            pltpu.async_copy(srcs_hbm.at[wid, j + 2],
                             idx_s.at[lax.rem(j + 2, 3)], isem)

        return carry

    lax.fori_loop(0, CH, chunk, 0)
    pltpu.make_async_copy(rows.at[0], acc.at[idx_d.at[0]],
                          ssem.at[(CH - 1) % 2]).wait()
    plsc.subcore_barrier()
    pltpu.sync_copy(acc.at[pl.ds(s * RPT, RPT)],
                    out_hbm.at[c, pl.ds(s * RPT, RPT)])


_seg_call = pl.kernel(
    _seg_body,
    out_type=jax.ShapeDtypeStruct((NC, N_PAD, D), jnp.float32),
    mesh=_MESH,
    scratch_types=[
        pltpu.VMEM((3, K), jnp.int32),
        pltpu.VMEM((CH, K), jnp.int32),
        pltpu.VMEM((2, K, D), jnp.float32),
        pltpu.VMEM_SHARED((N_PAD, D), jnp.float32),
        pltpu.SemaphoreType.DMA,
        pltpu.SemaphoreType.DMA,
        pltpu.SemaphoreType.DMA((2,)),
    ],
)


def _tc_dis_body(deg_ref, x_ref, w_ref, dis_ref, g_ref):
    deg = deg_ref[0, :, 0] + deg_ref[1, :, 0]
    dis = jnp.where(deg > 0, lax.rsqrt(deg), 0.0)[:, None]
    dis_ref[...] = jnp.broadcast_to(dis, (BLK, D))
    g_ref[...] = jnp.dot(x_ref[...], w_ref[...],
                         preferred_element_type=jnp.float32) * dis


_tc_dis = pl.pallas_call(
    _tc_dis_body,
    grid=(N_PAD // BLK,),
    in_specs=[
        pl.BlockSpec((NC, BLK, DEGW), lambda i: (0, i, 0)),
        pl.BlockSpec((BLK, D), lambda i: (i, 0)),
        pl.BlockSpec((D, D), lambda i: (0, 0)),
    ],
    out_specs=[
        pl.BlockSpec((BLK, D), lambda i: (i, 0)),
        pl.BlockSpec((BLK, D), lambda i: (i, 0)),
    ],
    out_shape=[
        jax.ShapeDtypeStruct((N_PAD, D), jnp.float32),
        jax.ShapeDtypeStruct((N_PAD, D), jnp.float32),
    ],
)


def _tc_b_body(acc_ref, dis_ref, b_ref, w_ref, g_ref):
    dis = dis_ref[...]
    h = jnp.maximum(dis * (acc_ref[0] + acc_ref[1]) + b_ref[...], 0.0)
    g_ref[...] = jnp.dot(h, w_ref[...],
                         preferred_element_type=jnp.float32) * dis


_tc_b = pl.pallas_call(
    _tc_b_body,
    grid=(N_PAD // BLK,),
    in_specs=[
        pl.BlockSpec((NC, BLK, D), lambda i: (0, i, 0)),
        pl.BlockSpec((BLK, D), lambda i: (i, 0)),
        pl.BlockSpec((D,), lambda i: (0,)),
        pl.BlockSpec((D, D), lambda i: (0, 0)),
    ],
    out_specs=pl.BlockSpec((BLK, D), lambda i: (i, 0)),
    out_shape=jax.ShapeDtypeStruct((N_PAD, D), jnp.float32),
)


def _tc_c_body(acc_ref, dis_ref, b_ref, w_ref, bfc_ref, out_ref):
    dis = dis_ref[...]
    h = jnp.maximum(dis * (acc_ref[0] + acc_ref[1]) + b_ref[...], 0.0)
    out_ref[...] = jnp.dot(h, w_ref[...],
                           preferred_element_type=jnp.float32) + bfc_ref[...][None, :]


_tc_c = pl.pallas_call(
    _tc_c_body,
    grid=(N_PAD // BLK,),
    in_specs=[
        pl.BlockSpec((NC, BLK, D), lambda i: (0, i, 0)),
        pl.BlockSpec((BLK, D), lambda i: (i, 0)),
        pl.BlockSpec((D,), lambda i: (0,)),
        pl.BlockSpec((D, D), lambda i: (0, 0)),
        pl.BlockSpec((D,), lambda i: (0,)),
    ],
    out_specs=pl.BlockSpec((BLK, D), lambda i: (i, 0)),
    out_shape=jax.ShapeDtypeStruct((N_PAD, D), jnp.float32),
)


def kernel(x, edge_index, W0, b0, W1, b1, W2, b2, Wfc, bfc):
    n = x.shape[0]
    idt = edge_index.dtype
    loop = jnp.arange(n, dtype=idt)
    pad = n + jnp.arange(E_PAD - E_TOT, dtype=idt) % (N_PAD - n)
    srcs = jnp.concatenate([edge_index[0], loop, pad]).reshape(NW, CH, K)
    dsts = jnp.concatenate([edge_index[1], loop, pad]).reshape(NW, CH, K)

    x_pad = jnp.zeros((N_PAD, D), jnp.float32).at[:n].set(x)
    zeros = jnp.zeros((RPT, D), jnp.float32)
    zeros_d = jnp.zeros((RPT, DEGW), jnp.float32)
    ones = jnp.ones((K, DEGW), jnp.float32)
    Wfc_p = jnp.zeros((D, D), jnp.float32).at[:, :D_OUT].set(Wfc)
    bfc_p = jnp.zeros((D,), jnp.float32).at[:D_OUT].set(bfc)

    deg2 = _deg_call(dsts, ones, zeros_d)
    dis, g = _tc_dis(deg2, x_pad, W0)
    acc = _seg_call(g, srcs, dsts, zeros)
    g = _tc_b(acc, dis, b0, W1)
    acc = _seg_call(g, srcs, dsts, zeros)
    g = _tc_b(acc, dis, b1, W2)
    acc = _seg_call(g, srcs, dsts, zeros)
    out = _tc_c(acc, dis, b2, Wfc_p, bfc_p)
    return out[:n, :D_OUT]


# speedup vs baseline: 5.3275x; 1.1563x over previous
import jax
import jax.numpy as jnp
from jax import lax
from jax.experimental import pallas as pl
from jax.experimental.pallas import tpu as pltpu
from jax.experimental.pallas import tpu_sc as plsc

N_NODES = 10000
D = 128
D_OUT = 40
E_RAW = 320000

NC, NS = 2, 16
NW = NC * NS
K = 128
E_TOT = E_RAW + N_NODES
CH = -(-E_TOT // (NW * K))
E_PAD = NW * CH * K
N_PAD = 10240
RPT = N_PAD // NS
DEGW = 16
BLK = 512
_MESH = plsc.VectorSubcoreMesh(core_axis_name="c", subcore_axis_name="s")


def _deg_body(dsts_hbm, ones_hbm, zeros_hbm, out_hbm, idx_d, ones_v, acc, dsem):
    c = lax.axis_index("c")
    s = lax.axis_index("s")
    wid = s * NC + c
    pltpu.sync_copy(dsts_hbm.at[wid], idx_d)
    pltpu.sync_copy(ones_hbm, ones_v)
    pltpu.sync_copy(zeros_hbm, acc.at[pl.ds(s * RPT, RPT)])
    plsc.subcore_barrier()

    def group(gi, carry):
        for k in range(8):
            pltpu.async_copy(ones_v, acc.at[idx_d.at[gi * 8 + k]], dsem,
                             add=True)
        for k in range(8):
            pltpu.make_async_copy(ones_v, acc.at[idx_d.at[0]], dsem).wait()
        return carry

    lax.fori_loop(0, CH // 8, group, 0)
    for j in range(CH - CH % 8, CH):
        pltpu.sync_copy(ones_v, acc.at[idx_d.at[j]], add=True)
    plsc.subcore_barrier()
    pltpu.sync_copy(acc.at[pl.ds(s * RPT, RPT)],
                    out_hbm.at[c, pl.ds(s * RPT, RPT)])


_deg_call = pl.kernel(
    _deg_body,
    out_type=jax.ShapeDtypeStruct((NC, N_PAD, DEGW), jnp.float32),
    mesh=_MESH,
    scratch_types=[
        pltpu.VMEM((CH, K), jnp.int32),
        pltpu.VMEM((K, DEGW), jnp.float32),
        pltpu.VMEM_SHARED((N_PAD, DEGW), jnp.float32),
        pltpu.SemaphoreType.DMA,
    ],
    compiler_params=pltpu.CompilerParams(use_tc_tiling_on_sc=False),
)


def _seg_body(g_hbm, srcs_hbm, dsts_hbm, zeros_hbm, out_hbm,
              idx_s, idx_d, rows, acc, gsem, isem, ssem):
    c = lax.axis_index("c")
    s = lax.axis_index("s")
    wid = s * NC + c
    pltpu.sync_copy(dsts_hbm.at[wid], idx_d)
    pltpu.sync_copy(srcs_hbm.at[wid, 0], idx_s.at[0])
    pltpu.async_copy(g_hbm.at[idx_s.at[0]], rows.at[0], gsem.at[0])
    pltpu.async_copy(srcs_hbm.at[wid, 1], idx_s.at[1], isem)
    pltpu.sync_copy(zeros_hbm, acc.at[pl.ds(s * RPT, RPT)])
    plsc.subcore_barrier()

    def chunk(j, carry):
        b = lax.rem(j, 2)

        @pl.when(j >= 1)
        def _():
            pltpu.make_async_copy(rows.at[0], acc.at[idx_d.at[0]],
                                  ssem.at[1 - b]).wait()

        @pl.when(j + 1 < CH)
        def _():
            pltpu.make_async_copy(srcs_hbm.at[wid, 0], idx_s.at[0], isem).wait()
            pltpu.async_copy(g_hbm.at[idx_s.at[lax.rem(j + 1, 3)]],
                             rows.at[1 - b], gsem.at[1 - b])

        @pl.when(j + 2 < CH)
        def _():
            pltpu.async_copy(srcs_hbm.at[wid, j + 2],
                             idx_s.at[lax.rem(j + 2, 3)], isem)

        pltpu.make_async_copy(g_hbm.at[idx_s.at[0]], rows.at[b],
                              gsem.at[b]).wait()
        pltpu.async_copy(rows.at[b], acc.at[idx_d.at[j]], ssem.at[b], add=True)
        return carry

    lax.fori_loop(0, CH, chunk, 0)
    pltpu.make_async_copy(rows.at[0], acc.at[idx_d.at[0]],
                          ssem.at[(CH - 1) % 2]).wait()
    plsc.subcore_barrier()
    pltpu.sync_copy(acc.at[pl.ds(s * RPT, RPT)],
                    out_hbm.at[c, pl.ds(s * RPT, RPT)])


_seg_call = pl.kernel(
    _seg_body,
    out_type=jax.ShapeDtypeStruct((NC, N_PAD, D), jnp.float32),
    mesh=_MESH,
    scratch_types=[
        pltpu.VMEM((3, K), jnp.int32),
        pltpu.VMEM((CH, K), jnp.int32),
        pltpu.VMEM((2, K, D), jnp.float32),
        pltpu.VMEM_SHARED((N_PAD, D), jnp.float32),
        pltpu.SemaphoreType.DMA((2,)),
        pltpu.SemaphoreType.DMA,
        pltpu.SemaphoreType.DMA((2,)),
    ],
)


def _tc_dis_body(deg_ref, x_ref, w_ref, dis_ref, g_ref):
    deg = deg_ref[0, :, 0] + deg_ref[1, :, 0]
    dis = jnp.where(deg > 0, lax.rsqrt(deg), 0.0)[:, None]
    dis_ref[...] = jnp.broadcast_to(dis, (BLK, D))
    g_ref[...] = jnp.dot(x_ref[...], w_ref[...],
                         preferred_element_type=jnp.float32) * dis


_tc_dis = pl.pallas_call(
    _tc_dis_body,
    grid=(N_PAD // BLK,),
    in_specs=[
        pl.BlockSpec((NC, BLK, DEGW), lambda i: (0, i, 0)),
        pl.BlockSpec((BLK, D), lambda i: (i, 0)),
        pl.BlockSpec((D, D), lambda i: (0, 0)),
    ],
    out_specs=[
        pl.BlockSpec((BLK, D), lambda i: (i, 0)),
        pl.BlockSpec((BLK, D), lambda i: (i, 0)),
    ],
    out_shape=[
        jax.ShapeDtypeStruct((N_PAD, D), jnp.float32),
        jax.ShapeDtypeStruct((N_PAD, D), jnp.float32),
    ],
)


def _tc_b_body(acc_ref, dis_ref, b_ref, w_ref, g_ref):
    dis = dis_ref[...]
    h = jnp.maximum(dis * (acc_ref[0] + acc_ref[1]) + b_ref[...], 0.0)
    g_ref[...] = jnp.dot(h, w_ref[...],
                         preferred_element_type=jnp.float32) * dis


_tc_b = pl.pallas_call(
    _tc_b_body,
    grid=(N_PAD // BLK,),
    in_specs=[
        pl.BlockSpec((NC, BLK, D), lambda i: (0, i, 0)),
        pl.BlockSpec((BLK, D), lambda i: (i, 0)),
        pl.BlockSpec((D,), lambda i: (0,)),
        pl.BlockSpec((D, D), lambda i: (0, 0)),
    ],
    out_specs=pl.BlockSpec((BLK, D), lambda i: (i, 0)),
    out_shape=jax.ShapeDtypeStruct((N_PAD, D), jnp.float32),
)


def _tc_c_body(acc_ref, dis_ref, b_ref, w_ref, bfc_ref, out_ref):
    dis = dis_ref[...]
    h = jnp.maximum(dis * (acc_ref[0] + acc_ref[1]) + b_ref[...], 0.0)
    out_ref[...] = jnp.dot(h, w_ref[...],
                           preferred_element_type=jnp.float32) + bfc_ref[...][None, :]


_tc_c = pl.pallas_call(
    _tc_c_body,
    grid=(N_PAD // BLK,),
    in_specs=[
        pl.BlockSpec((NC, BLK, D), lambda i: (0, i, 0)),
        pl.BlockSpec((BLK, D), lambda i: (i, 0)),
        pl.BlockSpec((D,), lambda i: (0,)),
        pl.BlockSpec((D, D), lambda i: (0, 0)),
        pl.BlockSpec((D,), lambda i: (0,)),
    ],
    out_specs=pl.BlockSpec((BLK, D), lambda i: (i, 0)),
    out_shape=jax.ShapeDtypeStruct((N_PAD, D), jnp.float32),
)


def kernel(x, edge_index, W0, b0, W1, b1, W2, b2, Wfc, bfc):
    n = x.shape[0]
    idt = edge_index.dtype
    loop = jnp.arange(n, dtype=idt)
    pad = n + jnp.arange(E_PAD - E_TOT, dtype=idt) % (N_PAD - n)
    srcs = jnp.concatenate([edge_index[0], loop, pad]).reshape(NW, CH, K)
    dsts = jnp.concatenate([edge_index[1], loop, pad]).reshape(NW, CH, K)

    x_pad = jnp.zeros((N_PAD, D), jnp.float32).at[:n].set(x)
    zeros = jnp.zeros((RPT, D), jnp.float32)
    zeros_d = jnp.zeros((RPT, DEGW), jnp.float32)
    ones = jnp.ones((K, DEGW), jnp.float32)
    Wfc_p = jnp.zeros((D, D), jnp.float32).at[:, :D_OUT].set(Wfc)
    bfc_p = jnp.zeros((D,), jnp.float32).at[:D_OUT].set(bfc)

    deg2 = _deg_call(dsts, ones, zeros_d)
    dis, g = _tc_dis(deg2, x_pad, W0)
    acc = _seg_call(g, srcs, dsts, zeros)
    g = _tc_b(acc, dis, b0, W1)
    acc = _seg_call(g, srcs, dsts, zeros)
    g = _tc_b(acc, dis, b1, W2)
    acc = _seg_call(g, srcs, dsts, zeros)
    out = _tc_c(acc, dis, b2, Wfc_p, bfc_p)
    return out[:n, :D_OUT]


# speedup vs baseline: 5.3311x; 1.0007x over previous
import jax
import jax.numpy as jnp
from jax import lax
from jax.experimental import pallas as pl
from jax.experimental.pallas import tpu as pltpu
from jax.experimental.pallas import tpu_sc as plsc

N_NODES = 10000
D = 128
D_OUT = 40
E_RAW = 320000

NC, NS = 2, 16
NW = NC * NS
K = 128
E_TOT = E_RAW + N_NODES
CH = -(-E_TOT // (NW * K))
E_PAD = NW * CH * K
N_PAD = 10240
RPT = N_PAD // NS
DEGW = 16
BLK = 512
_MESH = plsc.VectorSubcoreMesh(core_axis_name="c", subcore_axis_name="s")


def _deg_body(dsts_hbm, ones_hbm, zeros_hbm, out_hbm, idx_d, ones_v, acc, dsem):
    c = lax.axis_index("c")
    s = lax.axis_index("s")
    wid = s * NC + c
    pltpu.sync_copy(dsts_hbm.at[wid], idx_d)
    pltpu.sync_copy(ones_hbm, ones_v)
    pltpu.sync_copy(zeros_hbm, acc.at[pl.ds(s * RPT, RPT)])
    plsc.subcore_barrier()

    def group(gi, carry):
        for k in range(8):
            pltpu.async_copy(ones_v, acc.at[idx_d.at[gi * 8 + k]], dsem,
                             add=True)
        for k in range(8):
            pltpu.make_async_copy(ones_v, acc.at[idx_d.at[0]], dsem).wait()
        return carry

    lax.fori_loop(0, CH // 8, group, 0)
    for j in range(CH - CH % 8, CH):
        pltpu.sync_copy(ones_v, acc.at[idx_d.at[j]], add=True)
    plsc.subcore_barrier()
    pltpu.sync_copy(acc.at[pl.ds(s * RPT, RPT)],
                    out_hbm.at[c, pl.ds(s * RPT, RPT)])


_deg_call = pl.kernel(
    _deg_body,
    out_type=jax.ShapeDtypeStruct((NC, N_PAD, DEGW), jnp.float32),
    mesh=_MESH,
    scratch_types=[
        pltpu.VMEM((CH, K), jnp.int32),
        pltpu.VMEM((K, DEGW), jnp.float32),
        pltpu.VMEM_SHARED((N_PAD, DEGW), jnp.float32),
        pltpu.SemaphoreType.DMA,
    ],
    compiler_params=pltpu.CompilerParams(use_tc_tiling_on_sc=False, skip_device_barrier=True),
)


def _seg_body(g_hbm, srcs_hbm, dsts_hbm, zeros_hbm, out_hbm,
              idx_s, idx_d, rows, acc, gsem, isem, ssem):
    c = lax.axis_index("c")
    s = lax.axis_index("s")
    wid = s * NC + c
    pltpu.sync_copy(dsts_hbm.at[wid], idx_d)
    pltpu.sync_copy(srcs_hbm.at[wid, 0], idx_s.at[0])
    pltpu.async_copy(g_hbm.at[idx_s.at[0]], rows.at[0], gsem.at[0])
    pltpu.async_copy(srcs_hbm.at[wid, 1], idx_s.at[1], isem)
    pltpu.sync_copy(zeros_hbm, acc.at[pl.ds(s * RPT, RPT)])
    plsc.subcore_barrier()

    def chunk(j, carry):
        b = lax.rem(j, 2)

        @pl.when(j >= 1)
        def _():
            pltpu.make_async_copy(rows.at[0], acc.at[idx_d.at[0]],
                                  ssem.at[1 - b]).wait()

        @pl.when(j + 1 < CH)
        def _():
            pltpu.make_async_copy(srcs_hbm.at[wid, 0], idx_s.at[0], isem).wait()
            pltpu.async_copy(g_hbm.at[idx_s.at[lax.rem(j + 1, 3)]],
                             rows.at[1 - b], gsem.at[1 - b])

        @pl.when(j + 2 < CH)
        def _():
            pltpu.async_copy(srcs_hbm.at[wid, j + 2],
                             idx_s.at[lax.rem(j + 2, 3)], isem)

        pltpu.make_async_copy(g_hbm.at[idx_s.at[0]], rows.at[b],
                              gsem.at[b]).wait()
        pltpu.async_copy(rows.at[b], acc.at[idx_d.at[j]], ssem.at[b], add=True)
        return carry

    lax.fori_loop(0, CH, chunk, 0)
    pltpu.make_async_copy(rows.at[0], acc.at[idx_d.at[0]],
                          ssem.at[(CH - 1) % 2]).wait()
    plsc.subcore_barrier()
    pltpu.sync_copy(acc.at[pl.ds(s * RPT, RPT)],
                    out_hbm.at[c, pl.ds(s * RPT, RPT)])


_seg_call = pl.kernel(
    _seg_body,
    out_type=jax.ShapeDtypeStruct((NC, N_PAD, D), jnp.float32),
    mesh=_MESH,
    scratch_types=[
        pltpu.VMEM((3, K), jnp.int32),
        pltpu.VMEM((CH, K), jnp.int32),
        pltpu.VMEM((2, K, D), jnp.float32),
        pltpu.VMEM_SHARED((N_PAD, D), jnp.float32),
        pltpu.SemaphoreType.DMA((2,)),
        pltpu.SemaphoreType.DMA,
        pltpu.SemaphoreType.DMA((2,)),
    ],
    compiler_params=pltpu.CompilerParams(skip_device_barrier=True),
)


def _tc_dis_body(deg_ref, x_ref, w_ref, dis_ref, g_ref):
    deg = deg_ref[0, :, 0] + deg_ref[1, :, 0]
    dis = jnp.where(deg > 0, lax.rsqrt(deg), 0.0)[:, None]
    dis_ref[...] = jnp.broadcast_to(dis, (BLK, D))
    g_ref[...] = jnp.dot(x_ref[...], w_ref[...],
                         preferred_element_type=jnp.float32) * dis


_tc_dis = pl.pallas_call(
    _tc_dis_body,
    grid=(N_PAD // BLK,),
    in_specs=[
        pl.BlockSpec((NC, BLK, DEGW), lambda i: (0, i, 0)),
        pl.BlockSpec((BLK, D), lambda i: (i, 0)),
        pl.BlockSpec((D, D), lambda i: (0, 0)),
    ],
    out_specs=[
        pl.BlockSpec((BLK, D), lambda i: (i, 0)),
        pl.BlockSpec((BLK, D), lambda i: (i, 0)),
    ],
    out_shape=[
        jax.ShapeDtypeStruct((N_PAD, D), jnp.float32),
        jax.ShapeDtypeStruct((N_PAD, D), jnp.float32),
    ],
)


def _tc_b_body(acc_ref, dis_ref, b_ref, w_ref, g_ref):
    dis = dis_ref[...]
    h = jnp.maximum(dis * (acc_ref[0] + acc_ref[1]) + b_ref[...], 0.0)
    g_ref[...] = jnp.dot(h, w_ref[...],
                         preferred_element_type=jnp.float32) * dis


_tc_b = pl.pallas_call(
    _tc_b_body,
    grid=(N_PAD // BLK,),
    in_specs=[
        pl.BlockSpec((NC, BLK, D), lambda i: (0, i, 0)),
        pl.BlockSpec((BLK, D), lambda i: (i, 0)),
        pl.BlockSpec((D,), lambda i: (0,)),
        pl.BlockSpec((D, D), lambda i: (0, 0)),
    ],
    out_specs=pl.BlockSpec((BLK, D), lambda i: (i, 0)),
    out_shape=jax.ShapeDtypeStruct((N_PAD, D), jnp.float32),
)


def _tc_c_body(acc_ref, dis_ref, b_ref, w_ref, bfc_ref, out_ref):
    dis = dis_ref[...]
    h = jnp.maximum(dis * (acc_ref[0] + acc_ref[1]) + b_ref[...], 0.0)
    out_ref[...] = jnp.dot(h, w_ref[...],
                           preferred_element_type=jnp.float32) + bfc_ref[...][None, :]


_tc_c = pl.pallas_call(
    _tc_c_body,
    grid=(N_PAD // BLK,),
    in_specs=[
        pl.BlockSpec((NC, BLK, D), lambda i: (0, i, 0)),
        pl.BlockSpec((BLK, D), lambda i: (i, 0)),
        pl.BlockSpec((D,), lambda i: (0,)),
        pl.BlockSpec((D, D), lambda i: (0, 0)),
        pl.BlockSpec((D,), lambda i: (0,)),
    ],
    out_specs=pl.BlockSpec((BLK, D), lambda i: (i, 0)),
    out_shape=jax.ShapeDtypeStruct((N_PAD, D), jnp.float32),
)


def kernel(x, edge_index, W0, b0, W1, b1, W2, b2, Wfc, bfc):
    n = x.shape[0]
    idt = edge_index.dtype
    loop = jnp.arange(n, dtype=idt)
    pad = n + jnp.arange(E_PAD - E_TOT, dtype=idt) % (N_PAD - n)
    srcs = jnp.concatenate([edge_index[0], loop, pad]).reshape(NW, CH, K)
    dsts = jnp.concatenate([edge_index[1], loop, pad]).reshape(NW, CH, K)

    x_pad = jnp.zeros((N_PAD, D), jnp.float32).at[:n].set(x)
    zeros = jnp.zeros((RPT, D), jnp.float32)
    zeros_d = jnp.zeros((RPT, DEGW), jnp.float32)
    ones = jnp.ones((K, DEGW), jnp.float32)
    Wfc_p = jnp.zeros((D, D), jnp.float32).at[:, :D_OUT].set(Wfc)
    bfc_p = jnp.zeros((D,), jnp.float32).at[:D_OUT].set(bfc)

    deg2 = _deg_call(dsts, ones, zeros_d)
    dis, g = _tc_dis(deg2, x_pad, W0)
    acc = _seg_call(g, srcs, dsts, zeros)
    g = _tc_b(acc, dis, b0, W1)
    acc = _seg_call(g, srcs, dsts, zeros)
    g = _tc_b(acc, dis, b1, W2)
    acc = _seg_call(g, srcs, dsts, zeros)
    out = _tc_c(acc, dis, b2, Wfc_p, bfc_p)
    return out[:n, :D_OUT]
